# Initial kernel scaffold; baseline (speedup 1.0000x reference)
import jax
import jax.numpy as jnp
from jax.experimental import pallas as pl


def kernel(x, params):
    raise NotImplementedError("write your pallas kernel here")


# speedup vs baseline: 3.0600x; 3.0600x over previous
import jax
import jax.numpy as jnp
from jax import lax
from jax.experimental import pallas as pl


def _fps_kernel_body(ptsT_ref, out_ref):
    N = ptsT_ref.shape[2]
    K = out_ref.shape[2]
    xyzT = ptsT_ref[0, 0:3, :]
    iotaN = lax.broadcasted_iota(jnp.int32, (1, N), 1)
    iotaK = lax.broadcasted_iota(jnp.int32, (K, 1), 0)

    def body(i, carry):
        min_d, far, idx = carry
        idx = jnp.where(iotaK == i, far, idx)
        oh = (iotaN == far).astype(jnp.float32)
        cen = jnp.sum(xyzT * oh, axis=1, keepdims=True)
        diff = xyzT - cen
        d = jnp.sum(diff * diff, axis=0, keepdims=True)
        min_d = jnp.minimum(min_d, d)
        m = jnp.max(min_d)
        far2 = jnp.min(jnp.where(min_d == m, iotaN, N)).astype(jnp.int32)
        return (min_d, far2, idx)

    _, _, idx = lax.fori_loop(
        0, K, body,
        (jnp.full((1, N), 1e10, jnp.float32), jnp.int32(0),
         jnp.zeros((K, 1), jnp.int32)),
    )
    oh_kn = (idx == iotaN).astype(jnp.float32)
    out_ref[0] = lax.dot_general(
        ptsT_ref[0], oh_kn, (((1,), (1,)), ((), ())),
        preferred_element_type=jnp.float32, precision=lax.Precision.HIGHEST)


def _fps_call(ptsT, K):
    B, C, N = ptsT.shape
    return pl.pallas_call(
        _fps_kernel_body,
        grid=(B,),
        in_specs=[pl.BlockSpec((1, C, N), lambda b: (b, 0, 0))],
        out_specs=pl.BlockSpec((1, C, K), lambda b: (b, 0, 0)),
        out_shape=jax.ShapeDtypeStruct((B, C, K), jnp.float32),
    )(ptsT)


def _make_ball_kernel(r2, kq, nlayers):
    def kern(sel_ref, ptsT_ref, pts_ref, *refs):
        out_ref = refs[-1]
        wrefs = refs[:-1]
        q = sel_ref[0]
        Kq = q.shape[0]
        N = ptsT_ref.shape[2]
        qx = q[:, 0:3]

        d2 = jnp.zeros((Kq, N), jnp.float32)
        for c in range(3):
            qc = q[:, c:c + 1]
            pc = ptsT_ref[0, c:c + 1, :]
            diff = qc - pc
            d2 = d2 + diff * diff

        mask = (d2 < r2).astype(jnp.float32)
        rank = mask
        sh = 1
        while sh < N:
            shifted = jnp.concatenate(
                [jnp.zeros((Kq, sh), jnp.float32), rank[:, :N - sh]], axis=1)
            rank = rank + shifted
            sh *= 2

        pts = pts_ref[0]
        maskb = mask > 0.0
        h = None
        for j in range(kq):
            selj = jnp.where(
                jnp.logical_and(maskb, rank == jnp.float32(j + 1)),
                1.0, 0.0)
            x = jnp.dot(selj, pts, preferred_element_type=jnp.float32, precision=lax.Precision.HIGHEST)
            for li in range(nlayers):
                W = wrefs[2 * li][...]
                b = wrefs[2 * li + 1][...]
                x = jnp.maximum(
                    jnp.dot(x, W, preferred_element_type=jnp.float32, precision=lax.Precision.HIGHEST) + b, 0.0)
            h = x if h is None else jnp.maximum(h, x)
        out_ref[0] = jnp.concatenate([qx, h], axis=1)

    return kern


def _ball_call(sel, ptsT, pts, mlp, r, kq, kq_block):
    B, K, C = sel.shape
    N = pts.shape[1]
    nlayers = len(mlp)
    c_out = 3 + mlp[-1][0].shape[1]
    wargs = []
    wspecs = []
    for (W, b) in mlp:
        wargs += [W, b.reshape(1, -1)]
        wspecs += [
            pl.BlockSpec(W.shape, lambda b_, q_: (0, 0)),
            pl.BlockSpec((1, b.shape[0]), lambda b_, q_: (0, 0)),
        ]
    grid = (B, K // kq_block)
    return pl.pallas_call(
        _make_ball_kernel(r * r, kq, nlayers),
        grid=grid,
        in_specs=[
            pl.BlockSpec((1, kq_block, C), lambda b_, q_: (b_, q_, 0)),
            pl.BlockSpec((1, C, N), lambda b_, q_: (b_, 0, 0)),
            pl.BlockSpec((1, N, C), lambda b_, q_: (b_, 0, 0)),
        ] + wspecs,
        out_specs=pl.BlockSpec((1, kq_block, c_out), lambda b_, q_: (b_, q_, 0)),
        out_shape=jax.ShapeDtypeStruct((B, K, c_out), jnp.float32),
    )(sel, ptsT, pts, *wargs)


def _make_interp_kernel(c1, nlayers, has_lin):
    def kern(p1_ref, p2T_ref, p2_ref, *refs):
        out_ref = refs[-1]
        wrefs = refs[:-1]
        p1 = p1_ref[0]
        N1b = p1.shape[0]
        N2 = p2T_ref.shape[2]
        qx = p1[:, 0:3]

        d2 = jnp.zeros((N1b, N2), jnp.float32)
        for c in range(3):
            qc = p1[:, c:c + 1]
            pc = p2T_ref[0, c:c + 1, :]
            diff = qc - pc
            d2 = d2 + diff * diff
        d = jnp.sqrt(jnp.maximum(d2, 0.0))

        iota = lax.broadcasted_iota(jnp.int32, (N1b, N2), 1)
        S = jnp.zeros((N1b, N2), jnp.float32)
        wsum = jnp.zeros((N1b, 1), jnp.float32)
        dcur = d
        for _ in range(3):
            m = jnp.min(dcur, axis=1, keepdims=True)
            eq = dcur == m
            idx = jnp.min(jnp.where(eq, iota, N2), axis=1, keepdims=True)
            oh = iota == idx
            w = 1.0 / (m + 1e-8)
            S = S + jnp.where(oh, w, 0.0)
            wsum = wsum + w
            dcur = jnp.where(oh, jnp.float32(1e30), dcur)
        S = S / wsum

        f2 = p2_ref[0][:, 3:]
        feat = jnp.dot(S, f2, preferred_element_type=jnp.float32, precision=lax.Precision.HIGHEST)
        if c1 > 3:
            x = jnp.concatenate([p1[:, 3:], feat], axis=1)
        else:
            x = feat
        nw = nlayers
        for li in range(nw):
            W = wrefs[2 * li][...]
            b = wrefs[2 * li + 1][...]
            x = jnp.maximum(
                jnp.dot(x, W, preferred_element_type=jnp.float32, precision=lax.Precision.HIGHEST) + b, 0.0)
        if has_lin:
            lw = wrefs[2 * nw][...]
            lb = wrefs[2 * nw + 1][...]
            out_ref[0] = jnp.dot(x, lw, preferred_element_type=jnp.float32, precision=lax.Precision.HIGHEST) + lb
        else:
            out_ref[0] = jnp.concatenate([qx, x], axis=1)

    return kern


def _interp_call(p1, p2, mlp, n1_block, lin=None):
    B, N1, C1 = p1.shape
    _, N2, C2 = p2.shape
    p2T = jnp.transpose(p2, (0, 2, 1))
    nlayers = len(mlp)
    wargs = []
    wspecs = []
    for (W, b) in mlp:
        wargs += [W, b.reshape(1, -1)]
        wspecs += [
            pl.BlockSpec(W.shape, lambda b_, q_: (0, 0)),
            pl.BlockSpec((1, b.shape[0]), lambda b_, q_: (0, 0)),
        ]
    if lin is not None:
        lw, lb = lin
        wargs += [lw, lb.reshape(1, -1)]
        wspecs += [
            pl.BlockSpec(lw.shape, lambda b_, q_: (0, 0)),
            pl.BlockSpec((1, lb.shape[0]), lambda b_, q_: (0, 0)),
        ]
        c_out = lin[0].shape[1]
    else:
        c_out = 3 + mlp[-1][0].shape[1]
    grid = (B, N1 // n1_block)
    return pl.pallas_call(
        _make_interp_kernel(C1, nlayers, lin is not None),
        grid=grid,
        in_specs=[
            pl.BlockSpec((1, n1_block, C1), lambda b_, q_: (b_, q_, 0)),
            pl.BlockSpec((1, C2, N2), lambda b_, q_: (b_, 0, 0)),
            pl.BlockSpec((1, N2, C2), lambda b_, q_: (b_, 0, 0)),
        ] + wspecs,
        out_specs=pl.BlockSpec((1, n1_block, c_out), lambda b_, q_: (b_, q_, 0)),
        out_shape=jax.ShapeDtypeStruct((B, N1, c_out), jnp.float32),
    )(p1, p2T, p2, *wargs)


def _abstraction_level(points, K, r, mlp, kq_block):
    ptsT = jnp.transpose(points, (0, 2, 1))
    selT = _fps_call(ptsT, K)
    sel = jnp.transpose(selT, (0, 2, 1))
    return _ball_call(sel, ptsT, points, mlp, r, 16, kq_block)


def kernel(x, params):
    s1 = _abstraction_level(x, 1024, 0.1, params['as1'], 128)
    s2 = _abstraction_level(s1, 256, 0.2, params['as2'], 256)
    s3 = _abstraction_level(s2, 64, 0.4, params['as3'], 64)
    s4 = _abstraction_level(s3, 16, 0.8, params['as4'], 16)
    p3 = _interp_call(s3, s4, params['fp1'], 64)
    p2 = _interp_call(s2, p3, params['fp2'], 256)
    p1 = _interp_call(s1, p2, params['fp3'], 1024)
    logits = _interp_call(
        x, p1, params['fp4'], 512,
        lin=(params['lin2_W'], params['lin2_b']))
    return (x, logits)
